# baseline (speedup 1.0000x reference)
import functools

import jax
import jax.numpy as jnp
from jax import lax
from jax.experimental import pallas as pl
from jax.experimental.pallas import tpu as pltpu
from jax.experimental.pallas import tpu_sc as plsc

_NC = 2
_NS = 16
_NW = _NC * _NS

_CHUNK = 512


def _gather_call(n_flat: int, dim: int):
    b_per_w = n_flat // _NW
    n_chunks = b_per_w // _CHUNK
    mesh = plsc.VectorSubcoreMesh(
        core_axis_name="c", subcore_axis_name="s", num_cores=_NC, num_subcores=_NS
    )

    @functools.partial(
        pl.kernel,
        out_type=jax.ShapeDtypeStruct((n_flat, dim), jnp.float32),
        mesh=mesh,
        compiler_params=pltpu.CompilerParams(use_tc_tiling_on_sc=False),
        scratch_types=[
            pltpu.VMEM((_CHUNK,), jnp.int32),
            pltpu.VMEM((_CHUNK, dim), jnp.float32),
            pltpu.SemaphoreType.DMA,
        ],
    )
    def grab(ids_hbm, table_hbm, out_hbm, idx_v, rows_v, sem):
        wid = lax.axis_index("s") * _NC + lax.axis_index("c")
        base = wid * b_per_w

        def body(i, _):
            off = base + i * _CHUNK
            pltpu.sync_copy(ids_hbm.at[pl.ds(off, _CHUNK)], idx_v)
            pltpu.async_copy(table_hbm.at[idx_v], rows_v, sem).wait()
            pltpu.sync_copy(rows_v, out_hbm.at[pl.ds(off, _CHUNK)])
            return 0

        lax.fori_loop(0, n_chunks, body, 0)

    return grab


def kernel(token_ids, table):
    batch, seq = token_ids.shape
    _, dim = table.shape
    n_flat = batch * seq
    flat_ids = token_ids.reshape(n_flat).astype(jnp.int32)
    out = _gather_call(n_flat, dim)(flat_ids, table)
    return out.reshape(batch, seq, dim)


# speedup vs baseline: 1.0480x; 1.0480x over previous
import functools

import jax
import jax.numpy as jnp
from jax import lax
from jax.experimental import pallas as pl
from jax.experimental.pallas import tpu as pltpu
from jax.experimental.pallas import tpu_sc as plsc

_NC = 2
_NS = 16
_NW = _NC * _NS

_CHUNK = 512


def _gather_call(n_flat: int, dim: int):
    b_per_w = n_flat // _NW
    n_chunks = b_per_w // _CHUNK
    n_groups = n_chunks // 2
    assert n_chunks % 2 == 0 and n_groups >= 2
    mesh = plsc.VectorSubcoreMesh(
        core_axis_name="c", subcore_axis_name="s", num_cores=_NC, num_subcores=_NS
    )

    @functools.partial(
        pl.kernel,
        out_type=jax.ShapeDtypeStruct((n_flat, dim), jnp.float32),
        mesh=mesh,
        compiler_params=pltpu.CompilerParams(use_tc_tiling_on_sc=False),
        scratch_types=[
            pltpu.VMEM((b_per_w,), jnp.int32),
            pltpu.VMEM((2, _CHUNK, dim), jnp.float32),
            pltpu.SemaphoreType.DMA,
            pltpu.SemaphoreType.DMA,
            pltpu.SemaphoreType.DMA,
            pltpu.SemaphoreType.DMA,
        ],
    )
    def grab(ids_hbm, table_hbm, out_hbm, idx_all, rows_v, gs0, gs1, ws0, ws1):
        wid = lax.axis_index("s") * _NC + lax.axis_index("c")
        base = wid * b_per_w
        pltpu.sync_copy(ids_hbm.at[pl.ds(base, b_per_w)], idx_all)

        gsem = (gs0, gs1)
        wsem = (ws0, ws1)

        def gather(c, b):
            return pltpu.make_async_copy(
                table_hbm.at[idx_all.at[pl.ds(c * _CHUNK, _CHUNK)]],
                rows_v.at[b],
                gsem[b],
            )

        def wback(c, b):
            return pltpu.make_async_copy(
                rows_v.at[b],
                out_hbm.at[pl.ds(base + c * _CHUNK, _CHUNK)],
                wsem[b],
            )

        gather(0, 0).start()
        gather(1, 1).start()
        gather(0, 0).wait()
        wback(0, 0).start()
        wback(0, 0).wait()
        gather(2, 0).start()
        gather(1, 1).wait()
        wback(1, 1).start()

        def body(gi, _):
            c0 = 2 * gi
            c1 = c0 + 1
            wback(c0 - 1, 1).wait()
            gather(c1, 1).start()
            gather(c0, 0).wait()
            wback(c0, 0).start()
            wback(c0, 0).wait()
            gather(c0 + 2, 0).start()
            gather(c1, 1).wait()
            wback(c1, 1).start()
            return 0

        lax.fori_loop(1, n_groups - 1, body, 0)

        cl0 = n_chunks - 2
        cl1 = n_chunks - 1
        wback(cl0 - 1, 1).wait()
        gather(cl1, 1).start()
        gather(cl0, 0).wait()
        wback(cl0, 0).start()
        wback(cl0, 0).wait()
        gather(cl1, 1).wait()
        wback(cl1, 1).start()
        wback(cl1, 1).wait()

    return grab


def kernel(token_ids, table):
    batch, seq = token_ids.shape
    _, dim = table.shape
    n_flat = batch * seq
    flat_ids = token_ids.reshape(n_flat).astype(jnp.int32)
    out = _gather_call(n_flat, dim)(flat_ids, table)
    return out.reshape(batch, seq, dim)
